# Initial kernel scaffold; baseline (speedup 1.0000x reference)
import jax
import jax.numpy as jnp
from jax.experimental import pallas as pl


def kernel(features_omics1, features_omics2, edge_index_spatial_omics1, edge_index_feature_omics1, edge_index_spatial_omics2, edge_index_feature_omics2, Wl1, Wr1, att1, b1, Wl2, Wr2, att2, b2, dec_w1, dec_w2, aw1, au1, aw2, au2, awc, auc):
    raise NotImplementedError("write your pallas kernel here")


# speedup vs baseline: 10.2182x; 10.2182x over previous
import functools

import jax
import jax.numpy as jnp
from jax import lax
from jax.experimental import pallas as pl
from jax.experimental.pallas import tpu as pltpu
from jax.experimental.pallas import tpu_sc as plsc

N = 50000
NPAD = 50176
D_IN = 128
D_OUT = 64
H = 32

NC, NS, L = 2, 16, 16
NW = NC * NS

E = 800000
ECONV = E + N
CA = 128
ECP = ((ECONV + NW * CA - 1) // (NW * CA)) * (NW * CA)
EPWA = ECP // NW
CHA = EPWA // CA
CB = 512
CHB = EPWA // CB
ESP = ((E + NS * CB - 1) // (NS * CB)) * (NS * CB)
EPWS = ESP // NS
CHS = EPWS // CB
STRIPE = NPAD // NS

_MESH = plsc.VectorSubcoreMesh(core_axis_name="c", subcore_axis_name="s",
                               num_cores=NC, num_subcores=NS)

F32 = jnp.float32
I32 = jnp.int32


def _zero16():
    return jnp.zeros((L,), F32)


def _zero_vmem_2d(ref, rows, cols):
    zv = _zero16()

    def body(i, _):
        r = i // (cols // L)
        k = i % (cols // L)
        ref[r, pl.ds(k * L, L)] = zv
        return 0

    lax.fori_loop(0, rows * (cols // L), body, 0)


def _zero_vmem_1d(ref, n):
    zv = _zero16()

    def body(i, _):
        ref[pl.ds(i * L, L)] = zv
        return 0

    lax.fori_loop(0, n // L, body, 0)


def _zero_stripe(accS, zro_v, s):
    def body(i, _):
        pltpu.sync_copy(zro_v, accS.at[pl.ds(s * STRIPE + i * 64, 64), :])
        return 0

    lax.fori_loop(0, STRIPE // 64, body, 0)


@functools.partial(
    pl.kernel,
    mesh=_MESH,
    compiler_params=pltpu.CompilerParams(needs_layout_passes=False, use_tc_tiling_on_sc=False),
    out_type=(
        jax.ShapeDtypeStruct((NC, NPAD, H), F32),
        jax.ShapeDtypeStruct((ECP,), F32),
    ),
    scratch_types=[
        pltpu.VMEM((8, 128), I32),
        pltpu.VMEM((8, 128), I32),
        pltpu.VMEM((CA, D_OUT), F32),
        pltpu.VMEM((CA, D_OUT), F32),
        pltpu.VMEM((CA, H), F32),
        pltpu.VMEM((CA,), F32),
        pltpu.VMEM((D_OUT,), F32),
        pltpu.VMEM((64, H), F32),
        pltpu.VMEM_SHARED((NPAD, H), F32),
        pltpu.SemaphoreType.DMA,
        pltpu.SemaphoreType.DMA,
    ],
)
def _sc_conv_pass_a(xl_hbm, xr_hbm, src_hbm, dst_hbm, att_hbm,
                    acc_o, w_o,
                    src_v, dst_v, xl_v, xr_v, out_v, w_v, att_v,
                    zro_v, accS, sem1, sem2):
    c = lax.axis_index("c")
    s = lax.axis_index("s")
    wid = c * NS + s

    _zero_vmem_2d(zro_v, 64, H)
    _zero_stripe(accS, zro_v, s)
    pltpu.sync_copy(att_hbm, att_v)
    a0 = att_v[pl.ds(0, L)]
    a1 = att_v[pl.ds(L, L)]
    a2 = att_v[pl.ds(2 * L, L)]
    a3 = att_v[pl.ds(3 * L, L)]
    plsc.subcore_barrier()

    base0 = wid * EPWA
    i16 = jnp.arange(L, dtype=I32)
    aseg = (a0, a1, a2, a3)

    def chunk(g, _):
        sbase = base0 + g * 1024
        srow = pl.multiple_of(sbase // 128, 8)
        pltpu.sync_copy(src_hbm.at[pl.ds(srow, 8), :], src_v)
        pltpu.sync_copy(dst_hbm.at[pl.ds(srow, 8), :], dst_v)
        for j2 in range(8):
            cp1 = pltpu.async_copy(xl_hbm.at[src_v.at[j2]], xl_v, sem1)
            cp2 = pltpu.async_copy(xr_hbm.at[dst_v.at[j2]], xr_v, sem2)
            cp1.wait()
            cp2.wait()

            def score(k, _):
                sregs = []
                for t in range(L):
                    e = k * L + t
                    acc = _zero16()
                    for seg in range(4):
                        o = seg * L
                        z = (xl_v[e, pl.ds(o, L)] + xr_v[e, pl.ds(o, L)])
                        acc = acc + aseg[seg] * jnp.maximum(z, 0.2 * z)
                    sregs.append(jnp.sum(acc))
                sv = _zero16()
                for t in range(L):
                    sv = jnp.where(i16 == t, jnp.broadcast_to(sregs[t], (L,)),
                                   sv)
                wv = jnp.exp(sv)
                w_v[pl.ds(k * L, L)] = wv
                for t in range(L):
                    e = k * L + t
                    w16 = jnp.exp(jnp.broadcast_to(sregs[t], (L,)))
                    out_v[e, pl.ds(0, L)] = w16 * xl_v[e, pl.ds(0, L)]
                    out_v[e, pl.ds(L, L)] = w16 * xl_v[e, pl.ds(L, L)]
                return 0

            lax.fori_loop(0, CA // L, score, 0)

            pltpu.sync_copy(out_v, accS.at[dst_v.at[j2]], add=True)
            pltpu.sync_copy(w_v, w_o.at[pl.ds(sbase + j2 * CA, CA)])
        return 0

    lax.fori_loop(0, EPWA // 1024, chunk, 0)
    plsc.subcore_barrier()
    pltpu.sync_copy(accS.at[pl.ds(s * STRIPE, STRIPE), :],
                    acc_o.at[c, pl.ds(s * STRIPE, STRIPE), :])


@functools.partial(
    pl.kernel,
    mesh=_MESH,
    compiler_params=pltpu.CompilerParams(needs_layout_passes=False, use_tc_tiling_on_sc=False),
    out_type=jax.ShapeDtypeStruct((NW, NPAD), F32),
    scratch_types=[
        pltpu.VMEM((8, 128), I32),
        pltpu.VMEM((1024,), F32),
        pltpu.VMEM((NPAD,), F32),
    ],
)
def _sc_conv_pass_c(dst_hbm, w_hbm, ws_o, dst_v, w_v, wpriv):
    c = lax.axis_index("c")
    s = lax.axis_index("s")
    wid = c * NS + s

    _zero_vmem_1d(wpriv, NPAD)
    base0 = wid * EPWA

    def chunk(g, _):
        sbase = base0 + g * 1024
        srow = pl.multiple_of(sbase // 128, 8)
        pltpu.sync_copy(dst_hbm.at[pl.ds(srow, 8), :], dst_v)
        pltpu.sync_copy(w_hbm.at[pl.ds(sbase, 1024)], w_v)

        def grp(k, _):
            dv = dst_v[k // 8, pl.ds((k % 8) * L, L)]
            wv = w_v[pl.ds(k * L, L)]
            plsc.addupdate_scatter(wpriv, [dv], wv)
            return 0

        lax.fori_loop(0, 1024 // L, grp, 0)
        return 0

    lax.fori_loop(0, EPWA // 1024, chunk, 0)
    pltpu.sync_copy(wpriv, ws_o.at[wid])


@functools.partial(
    pl.kernel,
    mesh=_MESH,
    compiler_params=pltpu.CompilerParams(needs_layout_passes=False, use_tc_tiling_on_sc=False),
    out_type=jax.ShapeDtypeStruct((NC, NPAD, H), F32),
    scratch_types=[
        pltpu.VMEM((8, 128), I32),
        pltpu.VMEM((8, 128), I32),
        pltpu.VMEM((8, 128), I32),
        pltpu.VMEM((CB,), F32),
        pltpu.VMEM((CB, H), F32),
        pltpu.VMEM((64, H), F32),
        pltpu.VMEM_SHARED((NPAD, H), F32),
        pltpu.SemaphoreType.DMA,
    ],
)
def _sc_conv_pass_b(xlh_hbm, src_hbm, dst_hbm, w_hbm,
                    acc_o,
                    src_v, dst_v, idx2_v, w_v, rows_v, zro_v, accS, sem1):
    c = lax.axis_index("c")
    s = lax.axis_index("s")
    wid = c * NS + s

    _zero_vmem_2d(zro_v, 64, H)
    _zero_stripe(accS, zro_v, s)
    plsc.subcore_barrier()

    base0 = wid * EPWA
    i16 = jnp.arange(L, dtype=I32)

    def chunk(g, _):
        sbase = base0 + g * 1024
        srow = pl.multiple_of(sbase // 128, 8)
        pltpu.sync_copy(src_hbm.at[pl.ds(srow, 8), :], src_v)
        pltpu.sync_copy(dst_hbm.at[pl.ds(srow, 8), :], dst_v)

        def mkidx(k, _):
            j = k // 8
            t = (k % 8) * L
            idx2_v[j, pl.ds(t, L)] = src_v[j, pl.ds(t, L)] * 2 + 1
            return 0

        lax.fori_loop(0, 1024 // L, mkidx, 0)

        for j2 in range(2):
            pltpu.sync_copy(w_hbm.at[pl.ds(sbase + j2 * CB, CB)], w_v)
            for j in range(CB // 128):
                pltpu.async_copy(xlh_hbm.at[idx2_v.at[4 * j2 + j]],
                                 rows_v.at[pl.ds(j * 128, 128)], sem1).wait()

            def outrow(k, _):
                wv = w_v[pl.ds(k * L, L)]
                for t in range(L):
                    e = k * L + t
                    w16 = jnp.broadcast_to(jnp.sum(
                        jnp.where(i16 == t, wv, _zero16())), (L,))
                    rows_v[e, pl.ds(0, L)] = w16 * rows_v[e, pl.ds(0, L)]
                    rows_v[e, pl.ds(L, L)] = w16 * rows_v[e, pl.ds(L, L)]
                return 0

            lax.fori_loop(0, CB // L, outrow, 0)

            for j in range(CB // 128):
                pltpu.sync_copy(rows_v.at[pl.ds(j * 128, 128)],
                                accS.at[dst_v.at[4 * j2 + j]], add=True)
        return 0

    lax.fori_loop(0, EPWA // 1024, chunk, 0)
    plsc.subcore_barrier()
    pltpu.sync_copy(accS.at[pl.ds(s * STRIPE, STRIPE), :],
                    acc_o.at[c, pl.ds(s * STRIPE, STRIPE), :])


@functools.partial(
    pl.kernel,
    mesh=_MESH,
    compiler_params=pltpu.CompilerParams(needs_layout_passes=False, use_tc_tiling_on_sc=False),
    out_type=(
        jax.ShapeDtypeStruct((4, NPAD, H), F32),
        jax.ShapeDtypeStruct((4, NPAD, H), F32),
    ),
    scratch_types=[
        pltpu.VMEM((8, 128), I32),
        pltpu.VMEM((8, 128), I32),
        pltpu.VMEM((8, 128), I32),
        pltpu.VMEM((CB, H), F32),
        pltpu.VMEM((64, H), F32),
        pltpu.VMEM_SHARED((NPAD, H), F32),
        pltpu.SemaphoreType.DMA,
    ],
)
def _sc_spmm(embc_hbm, embx1_hbm, embx2_hbm,
             row1_hbm, col1_hbm, row2_hbm, col2_hbm,
             s1_o, s2_o,
             row_v, col_v, idx2_v, rows_v, zro_v, accS, sem1):
    c = lax.axis_index("c")
    s = lax.axis_index("s")

    _zero_vmem_2d(zro_v, 64, H)

    def do_edge_set(row_hbm, col_hbm, tables, out):
        base0 = s * EPWS
        for p, (tbl, h) in enumerate(tables):
            _zero_stripe(accS, zro_v, s)
            plsc.subcore_barrier()

            def chunk(g, _):
                sbase = base0 + g * 1024
                srow = pl.multiple_of(sbase // 128, 8)
                pltpu.sync_copy(row_hbm.at[pl.ds(srow, 8), :], row_v)
                pltpu.sync_copy(col_hbm.at[pl.ds(srow, 8), :], col_v)

                def mkidx(k, _):
                    j = k // 8
                    t = (k % 8) * L
                    idx2_v[j, pl.ds(t, L)] = col_v[j, pl.ds(t, L)] * 2 + h
                    return 0

                lax.fori_loop(0, 1024 // L, mkidx, 0)
                for j2 in range(2):
                    for j in range(CB // 128):
                        pltpu.async_copy(tbl.at[idx2_v.at[4 * j2 + j]],
                                         rows_v.at[pl.ds(j * 128, 128)],
                                         sem1).wait()
                    for j in range(CB // 128):
                        pltpu.sync_copy(rows_v.at[pl.ds(j * 128, 128)],
                                        accS.at[row_v.at[4 * j2 + j]],
                                        add=True)
                return 0

            lax.fori_loop(0, EPWS // 1024, chunk, 0)
            plsc.subcore_barrier()
            pltpu.sync_copy(accS.at[pl.ds(s * STRIPE, STRIPE), :],
                            out.at[p, pl.ds(s * STRIPE, STRIPE), :])
            plsc.subcore_barrier()

    @pl.when(c == 0)
    def _():
        do_edge_set(row1_hbm, col1_hbm,
                    [(embc_hbm, 0), (embc_hbm, 1),
                     (embx1_hbm, 0), (embx1_hbm, 1)], s1_o)

    @pl.when(c == 1)
    def _():
        do_edge_set(row2_hbm, col2_hbm,
                    [(embc_hbm, 0), (embc_hbm, 1),
                     (embx2_hbm, 0), (embx2_hbm, 1)], s2_o)


BN = 512
GRID = NPAD // BN


def _full(shape):
    return pl.BlockSpec(shape, lambda i: tuple(0 for _ in shape))


def _blk(shape, dim=0):
    def idx(i):
        return tuple(i if d == dim else 0 for d in range(len(shape)))
    return pl.BlockSpec(shape, idx)


def _tc1_body(f1, f2, wl1, wr1, wl2, wr2, xl1, xr1, xl2, xr2):
    a = f1[...]
    b = f2[...]
    xl1[...] = jnp.dot(a, wl1[...], preferred_element_type=F32)
    xr1[...] = jnp.dot(a, wr1[...], preferred_element_type=F32)
    xl2[...] = jnp.dot(b, wl2[...], preferred_element_type=F32)
    xr2[...] = jnp.dot(b, wr2[...], preferred_element_type=F32)


def _tc_xlxr(f1p, f2p, Wl1, Wr1, Wl2, Wr2):
    out = jax.ShapeDtypeStruct((NPAD, D_OUT), F32)
    return pl.pallas_call(
        _tc1_body,
        grid=(GRID,),
        in_specs=[_blk((BN, D_IN)), _blk((BN, D_IN)),
                  _full((D_IN, D_OUT)), _full((D_IN, D_OUT)),
                  _full((D_IN, D_OUT)), _full((D_IN, D_OUT))],
        out_specs=[_blk((BN, D_OUT))] * 4,
        out_shape=[out] * 4,
    )(f1p, f2p, Wl1, Wr1, Wl2, Wr2)


def _fin(accA, accB, ws, bias):
    a = accA[...]
    b = accB[...]
    den = jnp.sum(ws[...], axis=0) + 1e-16
    num = jnp.concatenate([a[0] + a[1], b[0] + b[1]], axis=-1)
    return num / den[:, None] + bias[...]


def _atten_blk(ea, eb, w_ref, u_ref):
    va = jnp.tanh(jnp.dot(ea, w_ref[...], preferred_element_type=F32))
    vb = jnp.tanh(jnp.dot(eb, w_ref[...], preferred_element_type=F32))
    u = u_ref[...]
    ua = jnp.dot(va, u, preferred_element_type=F32)[:, 0]
    ub = jnp.dot(vb, u, preferred_element_type=F32)[:, 0]
    m = jnp.maximum(ua, ub)
    x0 = jnp.exp(ua - m)
    x1 = jnp.exp(ub - m)
    den = x0 + x1
    a0 = x0 / den
    a1 = x1 / den
    comb = ea * a0[:, None] + eb * a1[:, None]
    return comb, jnp.stack([a0, a1], axis=1)


def _tc2_body(aA_sp1, aB_sp1, ws_sp1, aA_ft1, aB_ft1, ws_ft1,
              aA_sp2, aB_sp2, ws_sp2, aA_ft2, aB_ft2, ws_ft2,
              b1, b2, aw1, au1, aw2, au2, awc, auc,
              o_sp1, o_ft1, o_sp2, o_ft2, o_e1, o_e2, o_ec,
              o_al1, o_al2, o_al12):
    e_sp1 = _fin(aA_sp1, aB_sp1, ws_sp1, b1)
    e_ft1 = _fin(aA_ft1, aB_ft1, ws_ft1, b1)
    e_sp2 = _fin(aA_sp2, aB_sp2, ws_sp2, b2)
    e_ft2 = _fin(aA_ft2, aB_ft2, ws_ft2, b2)
    emb1, al1 = _atten_blk(e_sp1, e_ft1, aw1, au1)
    emb2, al2 = _atten_blk(e_sp2, e_ft2, aw2, au2)
    embc, al12 = _atten_blk(emb1, emb2, awc, auc)
    o_sp1[...] = e_sp1
    o_ft1[...] = e_ft1
    o_sp2[...] = e_sp2
    o_ft2[...] = e_ft2
    o_e1[...] = emb1
    o_e2[...] = emb2
    o_ec[...] = embc
    o_al1[...] = al1
    o_al2[...] = al2
    o_al12[...] = al12


def _tc_finalize1(aA_sp1, aB_sp1, ws_sp1, aA_ft1, aB_ft1, ws_ft1,
                  aA_sp2, aB_sp2, ws_sp2, aA_ft2, aB_ft2, ws_ft2,
                  b1, b2, aw1, au1, aw2, au2, awc, auc):
    acc_spec = _blk((NC, BN, H), dim=1)
    ws_spec = _blk((NW, BN), dim=1)
    emb = jax.ShapeDtypeStruct((NPAD, D_OUT), F32)
    al = jax.ShapeDtypeStruct((NPAD, 2), F32)
    return pl.pallas_call(
        _tc2_body,
        grid=(GRID,),
        in_specs=[acc_spec, acc_spec, ws_spec] * 4
        + [_full((1, D_OUT))] * 2
        + [_full((D_OUT, D_OUT)), _full((D_OUT, 1))] * 3,
        out_specs=[_blk((BN, D_OUT))] * 7 + [_blk((BN, 2))] * 3,
        out_shape=[emb] * 7 + [al] * 3,
    )(aA_sp1, aB_sp1, ws_sp1, aA_ft1, aB_ft1, ws_ft1,
      aA_sp2, aB_sp2, ws_sp2, aA_ft2, aB_ft2, ws_ft2,
      b1, b2, aw1, au1, aw2, au2, awc, auc)


def _tc3_body(s1, s2, dw1, dw2, wl1, wr1, wl2, wr2,
              o_rec1, o_rec2, o_xla1, o_xra1, o_xla2, o_xra2):
    t1 = s1[...]
    t2 = s2[...]
    sA = jnp.concatenate([t1[0], t1[1]], axis=-1)
    sB = jnp.concatenate([t1[2], t1[3]], axis=-1)
    sC = jnp.concatenate([t2[0], t2[1]], axis=-1)
    sD = jnp.concatenate([t2[2], t2[3]], axis=-1)
    w1 = dw1[...]
    w2 = dw2[...]
    o_rec1[...] = jnp.dot(sA, w1, preferred_element_type=F32)
    o_rec2[...] = jnp.dot(sC, w2, preferred_element_type=F32)
    x_a1 = jnp.dot(sD, w2, preferred_element_type=F32)
    x_a2 = jnp.dot(sB, w1, preferred_element_type=F32)
    o_xla1[...] = jnp.dot(x_a1, wl2[...], preferred_element_type=F32)
    o_xra1[...] = jnp.dot(x_a1, wr2[...], preferred_element_type=F32)
    o_xla2[...] = jnp.dot(x_a2, wl1[...], preferred_element_type=F32)
    o_xra2[...] = jnp.dot(x_a2, wr1[...], preferred_element_type=F32)


def _tc_stage3(s1, s2, dec_w1, dec_w2, Wl1, Wr1, Wl2, Wr2):
    rec = jax.ShapeDtypeStruct((NPAD, D_IN), F32)
    tab = jax.ShapeDtypeStruct((NPAD, D_OUT), F32)
    return pl.pallas_call(
        _tc3_body,
        grid=(GRID,),
        in_specs=[_blk((4, BN, H), dim=1), _blk((4, BN, H), dim=1),
                  _full((D_OUT, D_IN)), _full((D_OUT, D_IN)),
                  _full((D_IN, D_OUT)), _full((D_IN, D_OUT)),
                  _full((D_IN, D_OUT)), _full((D_IN, D_OUT))],
        out_specs=[_blk((BN, D_IN))] * 2 + [_blk((BN, D_OUT))] * 4,
        out_shape=[rec] * 2 + [tab] * 4,
    )(s1, s2, dec_w1, dec_w2, Wl1, Wr1, Wl2, Wr2)


def _tc4_body(aA1, aB1, ws1, aA2, aB2, ws2, b2, b1, o_a1, o_a2):
    o_a1[...] = _fin(aA1, aB1, ws1, b2)
    o_a2[...] = _fin(aA2, aB2, ws2, b1)


def _tc_finalize2(aA1, aB1, ws1, aA2, aB2, ws2, b2, b1):
    acc_spec = _blk((NC, BN, H), dim=1)
    ws_spec = _blk((NW, BN), dim=1)
    emb = jax.ShapeDtypeStruct((NPAD, D_OUT), F32)
    return pl.pallas_call(
        _tc4_body,
        grid=(GRID,),
        in_specs=[acc_spec, acc_spec, ws_spec] * 2 + [_full((1, D_OUT))] * 2,
        out_specs=[_blk((BN, D_OUT))] * 2,
        out_shape=[emb] * 2,
    )(aA1, aB1, ws1, aA2, aB2, ws2, b2, b1)


def _prep_conv_edges(e):
    loop = jnp.arange(N, dtype=I32)
    npad_e = ECP - ECONV
    src = jnp.concatenate([e[0], loop, jnp.zeros((npad_e,), I32)])
    dst = jnp.concatenate([e[1], loop, jnp.full((npad_e,), N, I32)])
    return src.reshape(ECP // 128, 128), dst.reshape(ECP // 128, 128)


def _prep_spmm_edges(e):
    npad_e = ESP - E
    row = jnp.concatenate([e[0], jnp.full((npad_e,), N, I32)])
    col = jnp.concatenate([e[1], jnp.zeros((npad_e,), I32)])
    return row.reshape(ESP // 128, 128), col.reshape(ESP // 128, 128)


def kernel(features_omics1, features_omics2, edge_index_spatial_omics1,
           edge_index_feature_omics1, edge_index_spatial_omics2,
           edge_index_feature_omics2, Wl1, Wr1, att1, b1, Wl2, Wr2, att2, b2,
           dec_w1, dec_w2, aw1, au1, aw2, au2, awc, auc):
    f1p = jnp.pad(features_omics1, ((0, NPAD - N), (0, 0)))
    f2p = jnp.pad(features_omics2, ((0, NPAD - N), (0, 0)))

    sp1_src, sp1_dst = _prep_conv_edges(edge_index_spatial_omics1)
    ft1_src, ft1_dst = _prep_conv_edges(edge_index_feature_omics1)
    sp2_src, sp2_dst = _prep_conv_edges(edge_index_spatial_omics2)
    ft2_src, ft2_dst = _prep_conv_edges(edge_index_feature_omics2)
    sp1_row, sp1_col = _prep_spmm_edges(edge_index_spatial_omics1)
    sp2_row, sp2_col = _prep_spmm_edges(edge_index_spatial_omics2)

    xl1, xr1, xl2, xr2 = _tc_xlxr(f1p, f2p, Wl1, Wr1, Wl2, Wr2)

    def conv(xl, xr, src, dst, att):
        accA, w = _sc_conv_pass_a(xl, xr, src, dst, att)
        accB = _sc_conv_pass_b(xl.reshape(2 * NPAD, H), src, dst, w)
        ws = _sc_conv_pass_c(dst, w)
        return accA, accB, ws

    aA_sp1, aB_sp1, ws_sp1 = conv(xl1, xr1, sp1_src, sp1_dst, att1)
    aA_ft1, aB_ft1, ws_ft1 = conv(xl1, xr1, ft1_src, ft1_dst, att1)
    aA_sp2, aB_sp2, ws_sp2 = conv(xl2, xr2, sp2_src, sp2_dst, att2)
    aA_ft2, aB_ft2, ws_ft2 = conv(xl2, xr2, ft2_src, ft2_dst, att2)

    (e_sp1, e_ft1, e_sp2, e_ft2, emb1, emb2, embc,
     al1, al2, al12) = _tc_finalize1(
        aA_sp1, aB_sp1, ws_sp1, aA_ft1, aB_ft1, ws_ft1,
        aA_sp2, aB_sp2, ws_sp2, aA_ft2, aB_ft2, ws_ft2,
        b1.reshape(1, D_OUT), b2.reshape(1, D_OUT),
        aw1, au1, aw2, au2, awc, auc)

    s1, s2 = _sc_spmm(embc.reshape(2 * NPAD, H), emb2.reshape(2 * NPAD, H),
                      emb1.reshape(2 * NPAD, H),
                      sp1_row, sp1_col, sp2_row, sp2_col)

    rec1, rec2, xla1, xra1, xla2, xra2 = _tc_stage3(
        s1, s2, dec_w1, dec_w2, Wl1, Wr1, Wl2, Wr2)

    aA_a1, aB_a1, ws_a1 = conv(xla1, xra1, sp2_src, sp2_dst, att2)
    aA_a2, aB_a2, ws_a2 = conv(xla2, xra2, sp1_src, sp1_dst, att1)

    across1, across2 = _tc_finalize2(aA_a1, aB_a1, ws_a1,
                                     aA_a2, aB_a2, ws_a2,
                                     b2.reshape(1, D_OUT),
                                     b1.reshape(1, D_OUT))

    return {
        'emb_latent_omics1': emb1[:N],
        'emb_latent_omics2': emb2[:N],
        'emb_latent_combined': embc[:N],
        'emb_recon_omics1': rec1[:N],
        'emb_recon_omics2': rec2[:N],
        'emb_latent_omics1_across_recon': across1[:N],
        'emb_latent_omics2_across_recon': across2[:N],
        'emb_latent_spatial_omics1': e_sp1[:N],
        'emb_latent_feature_omics1': e_ft1[:N],
        'emb_latent_spatial_omics2': e_sp2[:N],
        'emb_latent_feature_omics2': e_ft2[:N],
        'alpha_omics1': al1[:N],
        'alpha_omics2': al2[:N],
        'alpha': al12[:N],
    }


# speedup vs baseline: 13.7683x; 1.3474x over previous
import functools

import jax
import jax.numpy as jnp
from jax import lax
from jax.experimental import pallas as pl
from jax.experimental.pallas import tpu as pltpu
from jax.experimental.pallas import tpu_sc as plsc

N = 50000
NPAD = 50176
D_IN = 128
D_OUT = 64
H = 32

NC, NS, L = 2, 16, 16
NW = NC * NS

E = 800000
ECONV = E + N
CA = 128
ECP = ((ECONV + NW * CA - 1) // (NW * CA)) * (NW * CA)
EPWA = ECP // NW
CHA = EPWA // CA
CB = 512
CHB = EPWA // CB
ESP = ((E + NS * CB - 1) // (NS * CB)) * (NS * CB)
EPWS = ESP // NS
CHS = EPWS // CB
STRIPE = NPAD // NS

_MESH = plsc.VectorSubcoreMesh(core_axis_name="c", subcore_axis_name="s",
                               num_cores=NC, num_subcores=NS)

F32 = jnp.float32
I32 = jnp.int32


def _zero16():
    return jnp.zeros((L,), F32)


def _zero_vmem_2d(ref, rows, cols):
    zv = _zero16()

    def body(i, _):
        r = i // (cols // L)
        k = i % (cols // L)
        ref[r, pl.ds(k * L, L)] = zv
        return 0

    lax.fori_loop(0, rows * (cols // L), body, 0)


def _zero_vmem_1d(ref, n):
    zv = _zero16()

    def body(i, _):
        ref[pl.ds(i * L, L)] = zv
        return 0

    lax.fori_loop(0, n // L, body, 0)


def _zero_stripe(accS, zro_v, s):
    def body(i, _):
        pltpu.sync_copy(zro_v, accS.at[pl.ds(s * STRIPE + i * 64, 64), :])
        return 0

    lax.fori_loop(0, STRIPE // 64, body, 0)


@functools.partial(
    pl.kernel,
    mesh=_MESH,
    compiler_params=pltpu.CompilerParams(needs_layout_passes=False, use_tc_tiling_on_sc=False),
    out_type=(
        jax.ShapeDtypeStruct((NC, NPAD, H), F32),
        jax.ShapeDtypeStruct((ECP,), F32),
    ),
    scratch_types=[
        pltpu.VMEM((8, 128), I32),
        pltpu.VMEM((8, 128), I32),
        pltpu.VMEM((2, 64, D_OUT), F32),
        pltpu.VMEM((2, 64, D_OUT), F32),
        pltpu.VMEM((2, 128, H), F32),
        pltpu.VMEM((2, 128), F32),
        pltpu.VMEM((D_OUT,), F32),
        pltpu.VMEM((64, H), F32),
        pltpu.VMEM_SHARED((NPAD, H), F32),
        pltpu.SemaphoreType.DMA,
        pltpu.SemaphoreType.DMA,
        pltpu.SemaphoreType.DMA,
        pltpu.SemaphoreType.DMA,
        pltpu.SemaphoreType.DMA,
        pltpu.SemaphoreType.DMA,
    ],
)
def _sc_conv_pass_a(xl_hbm, xr_hbm, src_hbm, dst_hbm, att_hbm,
                    acc_o, w_o,
                    src_v, dst_v, xl_v, xr_v, out_v, w_v, att_v,
                    zro_v, accS, semg0, semg1, semsc0, semsc1, semw0, semw1):
    c = lax.axis_index("c")
    s = lax.axis_index("s")
    wid = c * NS + s

    _zero_vmem_2d(zro_v, 64, H)
    _zero_stripe(accS, zro_v, s)
    pltpu.sync_copy(att_hbm, att_v)
    a0 = att_v[pl.ds(0, L)]
    a1 = att_v[pl.ds(L, L)]
    a2 = att_v[pl.ds(2 * L, L)]
    a3 = att_v[pl.ds(3 * L, L)]
    plsc.subcore_barrier()

    base0 = wid * EPWA
    i16 = jnp.arange(L, dtype=I32)
    aseg = (a0, a1, a2, a3)
    semg = (semg0, semg1)
    semsc = (semsc0, semsc1)
    semw = (semw0, semw1)

    def issue_gathers(mc, p):
        q, hlf = mc // 2, (mc & 1) * 64
        cp1 = pltpu.async_copy(xl_hbm.at[src_v.at[q, pl.ds(hlf, 64)]],
                               xl_v.at[p], semg[p])
        cp2 = pltpu.async_copy(xr_hbm.at[dst_v.at[q, pl.ds(hlf, 64)]],
                               xr_v.at[p], semg[p])
        return (cp1, cp2)

    def chunk(g, _):
        sbase = base0 + g * 1024
        srow = pl.multiple_of(sbase // 128, 8)
        pltpu.sync_copy(src_hbm.at[pl.ds(srow, 8), :], src_v)
        pltpu.sync_copy(dst_hbm.at[pl.ds(srow, 8), :], dst_v)

        pend_g = [None, None]
        pend_sc = [None, None]
        pend_w = [None, None]
        pend_g[0] = issue_gathers(0, 0)

        for mc in range(16):
            p = mc & 1
            q = mc // 2
            p128 = q & 1
            hlf = (mc & 1) * 64
            for d in pend_g[p]:
                d.wait()
            if mc < 15:
                pend_g[1 - p] = issue_gathers(mc + 1, 1 - p)
            if (mc & 1) == 0:
                if pend_sc[p128] is not None:
                    pend_sc[p128].wait()
                    pend_sc[p128] = None
                if pend_w[p128] is not None:
                    pend_w[p128].wait()
                    pend_w[p128] = None

            def score(k, _):
                sregs = []
                for t in range(L):
                    e = k * L + t
                    acc = _zero16()
                    for seg in range(4):
                        o = seg * L
                        z = (xl_v[p, e, pl.ds(o, L)]
                             + xr_v[p, e, pl.ds(o, L)])
                        acc = acc + aseg[seg] * jnp.maximum(z, 0.2 * z)
                    sregs.append(jnp.sum(acc))
                sv = _zero16()
                for t in range(L):
                    sv = jnp.where(i16 == t, jnp.broadcast_to(sregs[t], (L,)),
                                   sv)
                wv = jnp.exp(sv)
                w_v[p128, pl.ds(hlf + k * L, L)] = wv
                for t in range(L):
                    e = k * L + t
                    w16 = jnp.exp(jnp.broadcast_to(sregs[t], (L,)))
                    out_v[p128, hlf + e, pl.ds(0, L)] = \
                        w16 * xl_v[p, e, pl.ds(0, L)]
                    out_v[p128, hlf + e, pl.ds(L, L)] = \
                        w16 * xl_v[p, e, pl.ds(L, L)]
                return 0

            lax.fori_loop(0, 4, score, 0)

            if mc & 1:
                pend_sc[p128] = pltpu.async_copy(
                    out_v.at[p128], accS.at[dst_v.at[q]], semsc[p128],
                    add=True)
                pend_w[p128] = pltpu.async_copy(
                    w_v.at[p128], w_o.at[pl.ds(sbase + q * 128, 128)],
                    semw[p128])
        for d in pend_sc + pend_w:
            if d is not None:
                d.wait()
        return 0

    lax.fori_loop(0, EPWA // 1024, chunk, 0)
    plsc.subcore_barrier()
    pltpu.sync_copy(accS.at[pl.ds(s * STRIPE, STRIPE), :],
                    acc_o.at[c, pl.ds(s * STRIPE, STRIPE), :])


@functools.partial(
    pl.kernel,
    mesh=_MESH,
    compiler_params=pltpu.CompilerParams(needs_layout_passes=False, use_tc_tiling_on_sc=False),
    out_type=jax.ShapeDtypeStruct((NW, NPAD), F32),
    scratch_types=[
        pltpu.VMEM((8, 128), I32),
        pltpu.VMEM((1024,), F32),
        pltpu.VMEM((NPAD,), F32),
    ],
)
def _sc_conv_pass_c(dst_hbm, w_hbm, ws_o, dst_v, w_v, wpriv):
    c = lax.axis_index("c")
    s = lax.axis_index("s")
    wid = c * NS + s

    _zero_vmem_1d(wpriv, NPAD)
    base0 = wid * EPWA

    def chunk(g, _):
        sbase = base0 + g * 1024
        srow = pl.multiple_of(sbase // 128, 8)
        pltpu.sync_copy(dst_hbm.at[pl.ds(srow, 8), :], dst_v)
        pltpu.sync_copy(w_hbm.at[pl.ds(sbase, 1024)], w_v)

        def grp(k, _):
            dv = dst_v[k // 8, pl.ds((k % 8) * L, L)]
            wv = w_v[pl.ds(k * L, L)]
            plsc.addupdate_scatter(wpriv, [dv], wv)
            return 0

        lax.fori_loop(0, 1024 // L, grp, 0)
        return 0

    lax.fori_loop(0, EPWA // 1024, chunk, 0)
    pltpu.sync_copy(wpriv, ws_o.at[wid])


@functools.partial(
    pl.kernel,
    mesh=_MESH,
    compiler_params=pltpu.CompilerParams(needs_layout_passes=False, use_tc_tiling_on_sc=False),
    out_type=jax.ShapeDtypeStruct((NC, NPAD, H), F32),
    scratch_types=[
        pltpu.VMEM((8, 128), I32),
        pltpu.VMEM((8, 128), I32),
        pltpu.VMEM((8, 128), I32),
        pltpu.VMEM((1024,), F32),
        pltpu.VMEM((2, 256, H), F32),
        pltpu.VMEM((64, H), F32),
        pltpu.VMEM_SHARED((NPAD, H), F32),
        pltpu.SemaphoreType.DMA,
        pltpu.SemaphoreType.DMA,
        pltpu.SemaphoreType.DMA,
        pltpu.SemaphoreType.DMA,
    ],
)
def _sc_conv_pass_b(xlh_hbm, src_hbm, dst_hbm, w_hbm,
                    acc_o,
                    src_v, dst_v, idx2_v, w_v, rows_v, zro_v, accS,
                    semg0, semg1, semsc0, semsc1):
    c = lax.axis_index("c")
    s = lax.axis_index("s")
    wid = c * NS + s

    _zero_vmem_2d(zro_v, 64, H)
    _zero_stripe(accS, zro_v, s)
    plsc.subcore_barrier()

    base0 = wid * EPWA
    i16 = jnp.arange(L, dtype=I32)
    semg = (semg0, semg1)
    semsc = (semsc0, semsc1)

    def issue_gathers(mc, p):
        return [pltpu.async_copy(xlh_hbm.at[idx2_v.at[2 * mc + j]],
                                 rows_v.at[p, pl.ds(j * 128, 128)], semg[p])
                for j in range(2)]

    def chunk(g, _):
        sbase = base0 + g * 1024
        srow = pl.multiple_of(sbase // 128, 8)
        pltpu.sync_copy(src_hbm.at[pl.ds(srow, 8), :], src_v)
        pltpu.sync_copy(dst_hbm.at[pl.ds(srow, 8), :], dst_v)
        pltpu.sync_copy(w_hbm.at[pl.ds(sbase, 1024)], w_v)

        def mkidx(k, _):
            j = k // 8
            t = (k % 8) * L
            idx2_v[j, pl.ds(t, L)] = src_v[j, pl.ds(t, L)] * 2 + 1
            return 0

        lax.fori_loop(0, 1024 // L, mkidx, 0)

        pend_g = [None, None]
        pend_sc = [None, None]
        pend_g[0] = issue_gathers(0, 0)
        for mc in range(4):
            p = mc & 1
            for d in pend_g[p]:
                d.wait()
            if pend_sc[1 - p] is not None:
                for d in pend_sc[1 - p]:
                    d.wait()
                pend_sc[1 - p] = None
            if mc < 3:
                pend_g[1 - p] = issue_gathers(mc + 1, 1 - p)

            def outrow(k, _):
                wv = w_v[pl.ds(mc * 256 + k * L, L)]
                for t in range(L):
                    e = k * L + t
                    w16 = jnp.broadcast_to(jnp.sum(
                        jnp.where(i16 == t, wv, _zero16())), (L,))
                    rows_v[p, e, pl.ds(0, L)] = \
                        w16 * rows_v[p, e, pl.ds(0, L)]
                    rows_v[p, e, pl.ds(L, L)] = \
                        w16 * rows_v[p, e, pl.ds(L, L)]
                return 0

            lax.fori_loop(0, 256 // L, outrow, 0)

            pend_sc[p] = [pltpu.async_copy(
                rows_v.at[p, pl.ds(j * 128, 128)],
                accS.at[dst_v.at[2 * mc + j]], semsc[p], add=True)
                for j in range(2)]
        for ds_ in pend_sc:
            if ds_ is not None:
                for d in ds_:
                    d.wait()
        return 0

    lax.fori_loop(0, EPWA // 1024, chunk, 0)
    plsc.subcore_barrier()
    pltpu.sync_copy(accS.at[pl.ds(s * STRIPE, STRIPE), :],
                    acc_o.at[c, pl.ds(s * STRIPE, STRIPE), :])


@functools.partial(
    pl.kernel,
    mesh=_MESH,
    compiler_params=pltpu.CompilerParams(needs_layout_passes=False, use_tc_tiling_on_sc=False),
    out_type=(
        jax.ShapeDtypeStruct((4, NPAD, H), F32),
        jax.ShapeDtypeStruct((4, NPAD, H), F32),
    ),
    scratch_types=[
        pltpu.VMEM((8, 128), I32),
        pltpu.VMEM((8, 128), I32),
        pltpu.VMEM((8, 128), I32),
        pltpu.VMEM((2, 256, H), F32),
        pltpu.VMEM((64, H), F32),
        pltpu.VMEM_SHARED((NPAD, H), F32),
        pltpu.SemaphoreType.DMA,
        pltpu.SemaphoreType.DMA,
        pltpu.SemaphoreType.DMA,
        pltpu.SemaphoreType.DMA,
    ],
)
def _sc_spmm(embc_hbm, embx1_hbm, embx2_hbm,
             row1_hbm, col1_hbm, row2_hbm, col2_hbm,
             s1_o, s2_o,
             row_v, col_v, idx2_v, rows_v, zro_v, accS,
             semg0, semg1, semsc0, semsc1):
    c = lax.axis_index("c")
    s = lax.axis_index("s")
    semg = (semg0, semg1)
    semsc = (semsc0, semsc1)

    _zero_vmem_2d(zro_v, 64, H)

    def do_edge_set(row_hbm, col_hbm, tables, out):
        base0 = s * EPWS
        for sp, (tbl, h) in enumerate(tables):
            _zero_stripe(accS, zro_v, s)
            plsc.subcore_barrier()

            def issue_gathers(mc, p):
                return [pltpu.async_copy(
                    tbl.at[idx2_v.at[2 * mc + j]],
                    rows_v.at[p, pl.ds(j * 128, 128)], semg[p])
                    for j in range(2)]

            def chunk(g, _):
                sbase = base0 + g * 1024
                srow = pl.multiple_of(sbase // 128, 8)
                pltpu.sync_copy(row_hbm.at[pl.ds(srow, 8), :], row_v)
                pltpu.sync_copy(col_hbm.at[pl.ds(srow, 8), :], col_v)

                def mkidx(k, _):
                    j = k // 8
                    t = (k % 8) * L
                    idx2_v[j, pl.ds(t, L)] = col_v[j, pl.ds(t, L)] * 2 + h
                    return 0

                lax.fori_loop(0, 1024 // L, mkidx, 0)

                pend_g = [None, None]
                pend_sc = [None, None]
                pend_g[0] = issue_gathers(0, 0)
                for mc in range(4):
                    p = mc & 1
                    for d in pend_g[p]:
                        d.wait()
                    if pend_sc[1 - p] is not None:
                        for d in pend_sc[1 - p]:
                            d.wait()
                        pend_sc[1 - p] = None
                    if mc < 3:
                        pend_g[1 - p] = issue_gathers(mc + 1, 1 - p)
                    pend_sc[p] = [pltpu.async_copy(
                        rows_v.at[p, pl.ds(j * 128, 128)],
                        accS.at[row_v.at[2 * mc + j]], semsc[p], add=True)
                        for j in range(2)]
                for ds_ in pend_sc:
                    if ds_ is not None:
                        for d in ds_:
                            d.wait()
                return 0

            lax.fori_loop(0, EPWS // 1024, chunk, 0)
            plsc.subcore_barrier()
            pltpu.sync_copy(accS.at[pl.ds(s * STRIPE, STRIPE), :],
                            out.at[sp, pl.ds(s * STRIPE, STRIPE), :])
            plsc.subcore_barrier()

    @pl.when(c == 0)
    def _():
        do_edge_set(row1_hbm, col1_hbm,
                    [(embc_hbm, 0), (embc_hbm, 1),
                     (embx1_hbm, 0), (embx1_hbm, 1)], s1_o)

    @pl.when(c == 1)
    def _():
        do_edge_set(row2_hbm, col2_hbm,
                    [(embc_hbm, 0), (embc_hbm, 1),
                     (embx2_hbm, 0), (embx2_hbm, 1)], s2_o)


BN = 512
GRID = NPAD // BN


def _full(shape):
    return pl.BlockSpec(shape, lambda i: tuple(0 for _ in shape))


def _blk(shape, dim=0):
    def idx(i):
        return tuple(i if d == dim else 0 for d in range(len(shape)))
    return pl.BlockSpec(shape, idx)


def _tc1_body(f1, f2, wl1, wr1, wl2, wr2, xl1, xr1, xl2, xr2):
    a = f1[...]
    b = f2[...]
    xl1[...] = jnp.dot(a, wl1[...], preferred_element_type=F32)
    xr1[...] = jnp.dot(a, wr1[...], preferred_element_type=F32)
    xl2[...] = jnp.dot(b, wl2[...], preferred_element_type=F32)
    xr2[...] = jnp.dot(b, wr2[...], preferred_element_type=F32)


def _tc_xlxr(f1p, f2p, Wl1, Wr1, Wl2, Wr2):
    out = jax.ShapeDtypeStruct((NPAD, D_OUT), F32)
    return pl.pallas_call(
        _tc1_body,
        grid=(GRID,),
        in_specs=[_blk((BN, D_IN)), _blk((BN, D_IN)),
                  _full((D_IN, D_OUT)), _full((D_IN, D_OUT)),
                  _full((D_IN, D_OUT)), _full((D_IN, D_OUT))],
        out_specs=[_blk((BN, D_OUT))] * 4,
        out_shape=[out] * 4,
    )(f1p, f2p, Wl1, Wr1, Wl2, Wr2)


def _fin(accA, accB, ws, bias):
    a = accA[...]
    b = accB[...]
    den = jnp.sum(ws[...], axis=0) + 1e-16
    num = jnp.concatenate([a[0] + a[1], b[0] + b[1]], axis=-1)
    return num / den[:, None] + bias[...]


def _atten_blk(ea, eb, w_ref, u_ref):
    va = jnp.tanh(jnp.dot(ea, w_ref[...], preferred_element_type=F32))
    vb = jnp.tanh(jnp.dot(eb, w_ref[...], preferred_element_type=F32))
    u = u_ref[...]
    ua = jnp.dot(va, u, preferred_element_type=F32)[:, 0]
    ub = jnp.dot(vb, u, preferred_element_type=F32)[:, 0]
    m = jnp.maximum(ua, ub)
    x0 = jnp.exp(ua - m)
    x1 = jnp.exp(ub - m)
    den = x0 + x1
    a0 = x0 / den
    a1 = x1 / den
    comb = ea * a0[:, None] + eb * a1[:, None]
    return comb, jnp.stack([a0, a1], axis=1)


def _tc2_body(aA_sp1, aB_sp1, ws_sp1, aA_ft1, aB_ft1, ws_ft1,
              aA_sp2, aB_sp2, ws_sp2, aA_ft2, aB_ft2, ws_ft2,
              b1, b2, aw1, au1, aw2, au2, awc, auc,
              o_sp1, o_ft1, o_sp2, o_ft2, o_e1, o_e2, o_ec,
              o_al1, o_al2, o_al12):
    e_sp1 = _fin(aA_sp1, aB_sp1, ws_sp1, b1)
    e_ft1 = _fin(aA_ft1, aB_ft1, ws_ft1, b1)
    e_sp2 = _fin(aA_sp2, aB_sp2, ws_sp2, b2)
    e_ft2 = _fin(aA_ft2, aB_ft2, ws_ft2, b2)
    emb1, al1 = _atten_blk(e_sp1, e_ft1, aw1, au1)
    emb2, al2 = _atten_blk(e_sp2, e_ft2, aw2, au2)
    embc, al12 = _atten_blk(emb1, emb2, awc, auc)
    o_sp1[...] = e_sp1
    o_ft1[...] = e_ft1
    o_sp2[...] = e_sp2
    o_ft2[...] = e_ft2
    o_e1[...] = emb1
    o_e2[...] = emb2
    o_ec[...] = embc
    o_al1[...] = al1
    o_al2[...] = al2
    o_al12[...] = al12


def _tc_finalize1(aA_sp1, aB_sp1, ws_sp1, aA_ft1, aB_ft1, ws_ft1,
                  aA_sp2, aB_sp2, ws_sp2, aA_ft2, aB_ft2, ws_ft2,
                  b1, b2, aw1, au1, aw2, au2, awc, auc):
    acc_spec = _blk((NC, BN, H), dim=1)
    ws_spec = _blk((NW, BN), dim=1)
    emb = jax.ShapeDtypeStruct((NPAD, D_OUT), F32)
    al = jax.ShapeDtypeStruct((NPAD, 2), F32)
    return pl.pallas_call(
        _tc2_body,
        grid=(GRID,),
        in_specs=[acc_spec, acc_spec, ws_spec] * 4
        + [_full((1, D_OUT))] * 2
        + [_full((D_OUT, D_OUT)), _full((D_OUT, 1))] * 3,
        out_specs=[_blk((BN, D_OUT))] * 7 + [_blk((BN, 2))] * 3,
        out_shape=[emb] * 7 + [al] * 3,
    )(aA_sp1, aB_sp1, ws_sp1, aA_ft1, aB_ft1, ws_ft1,
      aA_sp2, aB_sp2, ws_sp2, aA_ft2, aB_ft2, ws_ft2,
      b1, b2, aw1, au1, aw2, au2, awc, auc)


def _tc3_body(s1, s2, dw1, dw2, wl1, wr1, wl2, wr2,
              o_rec1, o_rec2, o_xla1, o_xra1, o_xla2, o_xra2):
    t1 = s1[...]
    t2 = s2[...]
    sA = jnp.concatenate([t1[0], t1[1]], axis=-1)
    sB = jnp.concatenate([t1[2], t1[3]], axis=-1)
    sC = jnp.concatenate([t2[0], t2[1]], axis=-1)
    sD = jnp.concatenate([t2[2], t2[3]], axis=-1)
    w1 = dw1[...]
    w2 = dw2[...]
    o_rec1[...] = jnp.dot(sA, w1, preferred_element_type=F32)
    o_rec2[...] = jnp.dot(sC, w2, preferred_element_type=F32)
    x_a1 = jnp.dot(sD, w2, preferred_element_type=F32)
    x_a2 = jnp.dot(sB, w1, preferred_element_type=F32)
    o_xla1[...] = jnp.dot(x_a1, wl2[...], preferred_element_type=F32)
    o_xra1[...] = jnp.dot(x_a1, wr2[...], preferred_element_type=F32)
    o_xla2[...] = jnp.dot(x_a2, wl1[...], preferred_element_type=F32)
    o_xra2[...] = jnp.dot(x_a2, wr1[...], preferred_element_type=F32)


def _tc_stage3(s1, s2, dec_w1, dec_w2, Wl1, Wr1, Wl2, Wr2):
    rec = jax.ShapeDtypeStruct((NPAD, D_IN), F32)
    tab = jax.ShapeDtypeStruct((NPAD, D_OUT), F32)
    return pl.pallas_call(
        _tc3_body,
        grid=(GRID,),
        in_specs=[_blk((4, BN, H), dim=1), _blk((4, BN, H), dim=1),
                  _full((D_OUT, D_IN)), _full((D_OUT, D_IN)),
                  _full((D_IN, D_OUT)), _full((D_IN, D_OUT)),
                  _full((D_IN, D_OUT)), _full((D_IN, D_OUT))],
        out_specs=[_blk((BN, D_IN))] * 2 + [_blk((BN, D_OUT))] * 4,
        out_shape=[rec] * 2 + [tab] * 4,
    )(s1, s2, dec_w1, dec_w2, Wl1, Wr1, Wl2, Wr2)


def _tc4_body(aA1, aB1, ws1, aA2, aB2, ws2, b2, b1, o_a1, o_a2):
    o_a1[...] = _fin(aA1, aB1, ws1, b2)
    o_a2[...] = _fin(aA2, aB2, ws2, b1)


def _tc_finalize2(aA1, aB1, ws1, aA2, aB2, ws2, b2, b1):
    acc_spec = _blk((NC, BN, H), dim=1)
    ws_spec = _blk((NW, BN), dim=1)
    emb = jax.ShapeDtypeStruct((NPAD, D_OUT), F32)
    return pl.pallas_call(
        _tc4_body,
        grid=(GRID,),
        in_specs=[acc_spec, acc_spec, ws_spec] * 2 + [_full((1, D_OUT))] * 2,
        out_specs=[_blk((BN, D_OUT))] * 2,
        out_shape=[emb] * 2,
    )(aA1, aB1, ws1, aA2, aB2, ws2, b2, b1)


def _prep_conv_edges(e):
    loop = jnp.arange(N, dtype=I32)
    npad_e = ECP - ECONV
    src = jnp.concatenate([e[0], loop, jnp.zeros((npad_e,), I32)])
    dst = jnp.concatenate([e[1], loop, jnp.full((npad_e,), N, I32)])
    return src.reshape(ECP // 128, 128), dst.reshape(ECP // 128, 128)


def _prep_spmm_edges(e):
    npad_e = ESP - E
    row = jnp.concatenate([e[0], jnp.full((npad_e,), N, I32)])
    col = jnp.concatenate([e[1], jnp.zeros((npad_e,), I32)])
    return row.reshape(ESP // 128, 128), col.reshape(ESP // 128, 128)


def kernel(features_omics1, features_omics2, edge_index_spatial_omics1,
           edge_index_feature_omics1, edge_index_spatial_omics2,
           edge_index_feature_omics2, Wl1, Wr1, att1, b1, Wl2, Wr2, att2, b2,
           dec_w1, dec_w2, aw1, au1, aw2, au2, awc, auc):
    f1p = jnp.pad(features_omics1, ((0, NPAD - N), (0, 0)))
    f2p = jnp.pad(features_omics2, ((0, NPAD - N), (0, 0)))

    sp1_src, sp1_dst = _prep_conv_edges(edge_index_spatial_omics1)
    ft1_src, ft1_dst = _prep_conv_edges(edge_index_feature_omics1)
    sp2_src, sp2_dst = _prep_conv_edges(edge_index_spatial_omics2)
    ft2_src, ft2_dst = _prep_conv_edges(edge_index_feature_omics2)
    sp1_row, sp1_col = _prep_spmm_edges(edge_index_spatial_omics1)
    sp2_row, sp2_col = _prep_spmm_edges(edge_index_spatial_omics2)

    xl1, xr1, xl2, xr2 = _tc_xlxr(f1p, f2p, Wl1, Wr1, Wl2, Wr2)

    def conv(xl, xr, src, dst, att):
        accA, w = _sc_conv_pass_a(xl, xr, src, dst, att)
        accB = _sc_conv_pass_b(xl.reshape(2 * NPAD, H), src, dst, w)
        ws = _sc_conv_pass_c(dst, w)
        return accA, accB, ws

    aA_sp1, aB_sp1, ws_sp1 = conv(xl1, xr1, sp1_src, sp1_dst, att1)
    aA_ft1, aB_ft1, ws_ft1 = conv(xl1, xr1, ft1_src, ft1_dst, att1)
    aA_sp2, aB_sp2, ws_sp2 = conv(xl2, xr2, sp2_src, sp2_dst, att2)
    aA_ft2, aB_ft2, ws_ft2 = conv(xl2, xr2, ft2_src, ft2_dst, att2)

    (e_sp1, e_ft1, e_sp2, e_ft2, emb1, emb2, embc,
     al1, al2, al12) = _tc_finalize1(
        aA_sp1, aB_sp1, ws_sp1, aA_ft1, aB_ft1, ws_ft1,
        aA_sp2, aB_sp2, ws_sp2, aA_ft2, aB_ft2, ws_ft2,
        b1.reshape(1, D_OUT), b2.reshape(1, D_OUT),
        aw1, au1, aw2, au2, awc, auc)

    s1, s2 = _sc_spmm(embc.reshape(2 * NPAD, H), emb2.reshape(2 * NPAD, H),
                      emb1.reshape(2 * NPAD, H),
                      sp1_row, sp1_col, sp2_row, sp2_col)

    rec1, rec2, xla1, xra1, xla2, xra2 = _tc_stage3(
        s1, s2, dec_w1, dec_w2, Wl1, Wr1, Wl2, Wr2)

    aA_a1, aB_a1, ws_a1 = conv(xla1, xra1, sp2_src, sp2_dst, att2)
    aA_a2, aB_a2, ws_a2 = conv(xla2, xra2, sp1_src, sp1_dst, att1)

    across1, across2 = _tc_finalize2(aA_a1, aB_a1, ws_a1,
                                     aA_a2, aB_a2, ws_a2,
                                     b2.reshape(1, D_OUT),
                                     b1.reshape(1, D_OUT))

    return {
        'emb_latent_omics1': emb1[:N],
        'emb_latent_omics2': emb2[:N],
        'emb_latent_combined': embc[:N],
        'emb_recon_omics1': rec1[:N],
        'emb_recon_omics2': rec2[:N],
        'emb_latent_omics1_across_recon': across1[:N],
        'emb_latent_omics2_across_recon': across2[:N],
        'emb_latent_spatial_omics1': e_sp1[:N],
        'emb_latent_feature_omics1': e_ft1[:N],
        'emb_latent_spatial_omics2': e_sp2[:N],
        'emb_latent_feature_omics2': e_ft2[:N],
        'alpha_omics1': al1[:N],
        'alpha_omics2': al2[:N],
        'alpha': al12[:N],
    }


# speedup vs baseline: 14.2576x; 1.0355x over previous
import functools

import jax
import jax.numpy as jnp
from jax import lax
from jax.experimental import pallas as pl
from jax.experimental.pallas import tpu as pltpu
from jax.experimental.pallas import tpu_sc as plsc

N = 50000
NPAD = 50176
D_IN = 128
D_OUT = 64
H = 32

NC, NS, L = 2, 16, 16
NW = NC * NS

E = 800000
ECONV = E + N
CA = 128
ECP = ((ECONV + NW * CA - 1) // (NW * CA)) * (NW * CA)
EPWA = ECP // NW
CHA = EPWA // CA
CB = 512
CHB = EPWA // CB
ESP = ((E + NS * CB - 1) // (NS * CB)) * (NS * CB)
EPWS = ESP // NS
CHS = EPWS // CB
STRIPE = NPAD // NS

_MESH = plsc.VectorSubcoreMesh(core_axis_name="c", subcore_axis_name="s",
                               num_cores=NC, num_subcores=NS)

F32 = jnp.float32
I32 = jnp.int32


def _zero16():
    return jnp.zeros((L,), F32)


def _zero_vmem_2d(ref, rows, cols):
    zv = _zero16()

    def body(i, _):
        r = i // (cols // L)
        k = i % (cols // L)
        ref[r, pl.ds(k * L, L)] = zv
        return 0

    lax.fori_loop(0, rows * (cols // L), body, 0)


def _zero_vmem_1d(ref, n):
    zv = _zero16()

    def body(i, _):
        ref[pl.ds(i * L, L)] = zv
        return 0

    lax.fori_loop(0, n // L, body, 0)


def _zero_stripe(accS, zro_v, s, rows=64):
    def body(i, _):
        pltpu.sync_copy(zro_v, accS.at[pl.ds(s * STRIPE + i * rows, rows), :])
        return 0

    lax.fori_loop(0, STRIPE // rows, body, 0)


@functools.partial(
    pl.kernel,
    mesh=_MESH,
    compiler_params=pltpu.CompilerParams(needs_layout_passes=False, use_tc_tiling_on_sc=False),
    out_type=(
        jax.ShapeDtypeStruct((NC, NPAD, H), F32),
        jax.ShapeDtypeStruct((ECP,), F32),
    ),
    scratch_types=[
        pltpu.VMEM((8, 128), I32),
        pltpu.VMEM((8, 128), I32),
        pltpu.VMEM((2, 64, D_OUT), F32),
        pltpu.VMEM((2, 64, D_OUT), F32),
        pltpu.VMEM((2, 128, H), F32),
        pltpu.VMEM((2, 128), F32),
        pltpu.VMEM((D_OUT,), F32),
        pltpu.VMEM((64, H), F32),
        pltpu.VMEM_SHARED((NPAD, H), F32),
        pltpu.SemaphoreType.DMA,
        pltpu.SemaphoreType.DMA,
        pltpu.SemaphoreType.DMA,
        pltpu.SemaphoreType.DMA,
        pltpu.SemaphoreType.DMA,
        pltpu.SemaphoreType.DMA,
    ],
)
def _sc_conv_pass_a(xl_hbm, xr_hbm, src_hbm, dst_hbm, att_hbm,
                    acc_o, w_o,
                    src_v, dst_v, xl_v, xr_v, out_v, w_v, att_v,
                    zro_v, accS, semg0, semg1, semsc0, semsc1, semw0, semw1):
    c = lax.axis_index("c")
    s = lax.axis_index("s")
    wid = c * NS + s

    _zero_vmem_2d(zro_v, 64, H)
    _zero_stripe(accS, zro_v, s)
    pltpu.sync_copy(att_hbm, att_v)
    a0 = att_v[pl.ds(0, L)]
    a1 = att_v[pl.ds(L, L)]
    a2 = att_v[pl.ds(2 * L, L)]
    a3 = att_v[pl.ds(3 * L, L)]
    plsc.subcore_barrier()

    base0 = wid * EPWA
    i16 = jnp.arange(L, dtype=I32)
    aseg = (a0, a1, a2, a3)
    semg = (semg0, semg1)
    semsc = (semsc0, semsc1)
    semw = (semw0, semw1)

    def issue_gathers(mc, p):
        q, hlf = mc // 2, (mc & 1) * 64
        cp1 = pltpu.async_copy(xl_hbm.at[src_v.at[q, pl.ds(hlf, 64)]],
                               xl_v.at[p], semg[p])
        cp2 = pltpu.async_copy(xr_hbm.at[dst_v.at[q, pl.ds(hlf, 64)]],
                               xr_v.at[p], semg[p])
        return (cp1, cp2)

    def chunk(g, _):
        sbase = base0 + g * 1024
        srow = pl.multiple_of(sbase // 128, 8)
        pltpu.sync_copy(src_hbm.at[pl.ds(srow, 8), :], src_v)
        pltpu.sync_copy(dst_hbm.at[pl.ds(srow, 8), :], dst_v)

        pend_g = [None, None]
        pend_sc = [None, None]
        pend_w = [None, None]
        pend_g[0] = issue_gathers(0, 0)

        for mc in range(16):
            p = mc & 1
            q = mc // 2
            p128 = q & 1
            hlf = (mc & 1) * 64
            for d in pend_g[p]:
                d.wait()
            if mc < 15:
                pend_g[1 - p] = issue_gathers(mc + 1, 1 - p)
            if (mc & 1) == 0:
                if pend_sc[p128] is not None:
                    pend_sc[p128].wait()
                    pend_sc[p128] = None
                if pend_w[p128] is not None:
                    pend_w[p128].wait()
                    pend_w[p128] = None

            def score(k, _):
                sregs = []
                for t in range(L):
                    e = k * L + t
                    acc = _zero16()
                    for seg in range(4):
                        o = seg * L
                        z = (xl_v[p, e, pl.ds(o, L)]
                             + xr_v[p, e, pl.ds(o, L)])
                        acc = acc + aseg[seg] * jnp.maximum(z, 0.2 * z)
                    sregs.append(jnp.sum(acc))
                sv = _zero16()
                for t in range(L):
                    sv = jnp.where(i16 == t, jnp.broadcast_to(sregs[t], (L,)),
                                   sv)
                wv = jnp.exp(sv)
                w_v[p128, pl.ds(hlf + k * L, L)] = wv
                for t in range(L):
                    e = k * L + t
                    w16 = jnp.exp(jnp.broadcast_to(sregs[t], (L,)))
                    out_v[p128, hlf + e, pl.ds(0, L)] = \
                        w16 * xl_v[p, e, pl.ds(0, L)]
                    out_v[p128, hlf + e, pl.ds(L, L)] = \
                        w16 * xl_v[p, e, pl.ds(L, L)]
                return 0

            lax.fori_loop(0, 4, score, 0)

            if mc & 1:
                pend_sc[p128] = pltpu.async_copy(
                    out_v.at[p128], accS.at[dst_v.at[q]], semsc[p128],
                    add=True)
                pend_w[p128] = pltpu.async_copy(
                    w_v.at[p128], w_o.at[pl.ds(sbase + q * 128, 128)],
                    semw[p128])
        for d in pend_sc + pend_w:
            if d is not None:
                d.wait()
        return 0

    lax.fori_loop(0, EPWA // 1024, chunk, 0)
    plsc.subcore_barrier()
    pltpu.sync_copy(accS.at[pl.ds(s * STRIPE, STRIPE), :],
                    acc_o.at[c, pl.ds(s * STRIPE, STRIPE), :])


@functools.partial(
    pl.kernel,
    mesh=_MESH,
    compiler_params=pltpu.CompilerParams(needs_layout_passes=False, use_tc_tiling_on_sc=False),
    out_type=jax.ShapeDtypeStruct((NW, NPAD), F32),
    scratch_types=[
        pltpu.VMEM((8, 128), I32),
        pltpu.VMEM((1024,), F32),
        pltpu.VMEM((NPAD,), F32),
    ],
)
def _sc_conv_pass_c(dst_hbm, w_hbm, ws_o, dst_v, w_v, wpriv):
    c = lax.axis_index("c")
    s = lax.axis_index("s")
    wid = c * NS + s

    _zero_vmem_1d(wpriv, NPAD)
    base0 = wid * EPWA

    def chunk(g, _):
        sbase = base0 + g * 1024
        srow = pl.multiple_of(sbase // 128, 8)
        pltpu.sync_copy(dst_hbm.at[pl.ds(srow, 8), :], dst_v)
        pltpu.sync_copy(w_hbm.at[pl.ds(sbase, 1024)], w_v)

        def grp(k, _):
            dv = dst_v[k // 8, pl.ds((k % 8) * L, L)]
            wv = w_v[pl.ds(k * L, L)]
            plsc.addupdate_scatter(wpriv, [dv], wv)
            return 0

        lax.fori_loop(0, 1024 // L, grp, 0)
        return 0

    lax.fori_loop(0, EPWA // 1024, chunk, 0)
    pltpu.sync_copy(wpriv, ws_o.at[wid])


@functools.partial(
    pl.kernel,
    mesh=_MESH,
    compiler_params=pltpu.CompilerParams(needs_layout_passes=False, use_tc_tiling_on_sc=False),
    out_type=jax.ShapeDtypeStruct((NC, NPAD, H), F32),
    scratch_types=[
        pltpu.VMEM((8, 128), I32),
        pltpu.VMEM((8, 128), I32),
        pltpu.VMEM((8, 128), I32),
        pltpu.VMEM((1024,), F32),
        pltpu.VMEM((3, 256, H), F32),
        pltpu.VMEM((32, H), F32),
        pltpu.VMEM_SHARED((NPAD, H), F32),
        pltpu.SemaphoreType.DMA,
        pltpu.SemaphoreType.DMA,
        pltpu.SemaphoreType.DMA,
        pltpu.SemaphoreType.DMA,
        pltpu.SemaphoreType.DMA,
        pltpu.SemaphoreType.DMA,
    ],
)
def _sc_conv_pass_b(xlh_hbm, src_hbm, dst_hbm, w_hbm,
                    acc_o,
                    src_v, dst_v, idx2_v, w_v, rows_v, zro_v, accS,
                    semg0, semg1, semg2, semsc0, semsc1, semsc2):
    c = lax.axis_index("c")
    s = lax.axis_index("s")
    wid = c * NS + s

    _zero_vmem_2d(zro_v, 32, H)
    _zero_stripe(accS, zro_v, s, rows=32)
    plsc.subcore_barrier()

    base0 = wid * EPWA
    i16 = jnp.arange(L, dtype=I32)
    semg = (semg0, semg1, semg2)
    semsc = (semsc0, semsc1, semsc2)

    def issue_gathers(mc, p):
        return [pltpu.async_copy(xlh_hbm.at[idx2_v.at[2 * mc + j]],
                                 rows_v.at[p, pl.ds(j * 128, 128)], semg[p])
                for j in range(2)]

    def chunk(g, _):
        sbase = base0 + g * 1024
        srow = pl.multiple_of(sbase // 128, 8)
        pltpu.sync_copy(src_hbm.at[pl.ds(srow, 8), :], src_v)
        pltpu.sync_copy(dst_hbm.at[pl.ds(srow, 8), :], dst_v)
        pltpu.sync_copy(w_hbm.at[pl.ds(sbase, 1024)], w_v)

        def mkidx(k, _):
            j = k // 8
            t = (k % 8) * L
            idx2_v[j, pl.ds(t, L)] = src_v[j, pl.ds(t, L)] * 2 + 1
            return 0

        lax.fori_loop(0, 1024 // L, mkidx, 0)

        pend_g = [None, None, None]
        pend_sc = [None, None, None]
        pend_g[0] = issue_gathers(0, 0)
        for mc in range(4):
            p = mc % 3
            np_ = (mc + 1) % 3
            for d in pend_g[p]:
                d.wait()
            if mc < 3:
                if pend_sc[np_] is not None:
                    for d in pend_sc[np_]:
                        d.wait()
                    pend_sc[np_] = None
                pend_g[np_] = issue_gathers(mc + 1, np_)

            def outrow(k, _):
                wv = w_v[pl.ds(mc * 256 + k * L, L)]
                for t in range(L):
                    e = k * L + t
                    w16 = jnp.broadcast_to(jnp.sum(
                        jnp.where(i16 == t, wv, _zero16())), (L,))
                    rows_v[p, e, pl.ds(0, L)] = \
                        w16 * rows_v[p, e, pl.ds(0, L)]
                    rows_v[p, e, pl.ds(L, L)] = \
                        w16 * rows_v[p, e, pl.ds(L, L)]
                return 0

            lax.fori_loop(0, 256 // L, outrow, 0)

            pend_sc[p] = [pltpu.async_copy(
                rows_v.at[p, pl.ds(j * 128, 128)],
                accS.at[dst_v.at[2 * mc + j]], semsc[p], add=True)
                for j in range(2)]
        for ds_ in pend_sc:
            if ds_ is not None:
                for d in ds_:
                    d.wait()
        return 0

    lax.fori_loop(0, EPWA // 1024, chunk, 0)
    plsc.subcore_barrier()
    pltpu.sync_copy(accS.at[pl.ds(s * STRIPE, STRIPE), :],
                    acc_o.at[c, pl.ds(s * STRIPE, STRIPE), :])


@functools.partial(
    pl.kernel,
    mesh=_MESH,
    compiler_params=pltpu.CompilerParams(needs_layout_passes=False, use_tc_tiling_on_sc=False),
    out_type=(
        jax.ShapeDtypeStruct((4, NPAD, H), F32),
        jax.ShapeDtypeStruct((4, NPAD, H), F32),
    ),
    scratch_types=[
        pltpu.VMEM((8, 128), I32),
        pltpu.VMEM((8, 128), I32),
        pltpu.VMEM((8, 128), I32),
        pltpu.VMEM((3, 256, H), F32),
        pltpu.VMEM((32, H), F32),
        pltpu.VMEM_SHARED((NPAD, H), F32),
        pltpu.SemaphoreType.DMA,
        pltpu.SemaphoreType.DMA,
        pltpu.SemaphoreType.DMA,
        pltpu.SemaphoreType.DMA,
        pltpu.SemaphoreType.DMA,
        pltpu.SemaphoreType.DMA,
    ],
)
def _sc_spmm(embc_hbm, embx1_hbm, embx2_hbm,
             row1_hbm, col1_hbm, row2_hbm, col2_hbm,
             s1_o, s2_o,
             row_v, col_v, idx2_v, rows_v, zro_v, accS,
             semg0, semg1, semg2, semsc0, semsc1, semsc2):
    c = lax.axis_index("c")
    s = lax.axis_index("s")
    semg = (semg0, semg1, semg2)
    semsc = (semsc0, semsc1, semsc2)

    _zero_vmem_2d(zro_v, 32, H)

    def do_edge_set(row_hbm, col_hbm, tables, out):
        base0 = s * EPWS
        for sp, (tbl, h) in enumerate(tables):
            _zero_stripe(accS, zro_v, s, rows=32)
            plsc.subcore_barrier()

            def issue_gathers(mc, p):
                return [pltpu.async_copy(
                    tbl.at[idx2_v.at[2 * mc + j]],
                    rows_v.at[p, pl.ds(j * 128, 128)], semg[p])
                    for j in range(2)]

            def chunk(g, _):
                sbase = base0 + g * 1024
                srow = pl.multiple_of(sbase // 128, 8)
                pltpu.sync_copy(row_hbm.at[pl.ds(srow, 8), :], row_v)
                pltpu.sync_copy(col_hbm.at[pl.ds(srow, 8), :], col_v)

                def mkidx(k, _):
                    j = k // 8
                    t = (k % 8) * L
                    idx2_v[j, pl.ds(t, L)] = col_v[j, pl.ds(t, L)] * 2 + h
                    return 0

                lax.fori_loop(0, 1024 // L, mkidx, 0)

                pend_g = [None, None, None]
                pend_sc = [None, None, None]
                pend_g[0] = issue_gathers(0, 0)
                for mc in range(4):
                    p = mc % 3
                    np_ = (mc + 1) % 3
                    for d in pend_g[p]:
                        d.wait()
                    if mc < 3:
                        if pend_sc[np_] is not None:
                            for d in pend_sc[np_]:
                                d.wait()
                            pend_sc[np_] = None
                        pend_g[np_] = issue_gathers(mc + 1, np_)
                    pend_sc[p] = [pltpu.async_copy(
                        rows_v.at[p, pl.ds(j * 128, 128)],
                        accS.at[row_v.at[2 * mc + j]], semsc[p], add=True)
                        for j in range(2)]
                for ds_ in pend_sc:
                    if ds_ is not None:
                        for d in ds_:
                            d.wait()
                return 0

            lax.fori_loop(0, EPWS // 1024, chunk, 0)
            plsc.subcore_barrier()
            pltpu.sync_copy(accS.at[pl.ds(s * STRIPE, STRIPE), :],
                            out.at[sp, pl.ds(s * STRIPE, STRIPE), :])
            plsc.subcore_barrier()

    @pl.when(c == 0)
    def _():
        do_edge_set(row1_hbm, col1_hbm,
                    [(embc_hbm, 0), (embc_hbm, 1),
                     (embx1_hbm, 0), (embx1_hbm, 1)], s1_o)

    @pl.when(c == 1)
    def _():
        do_edge_set(row2_hbm, col2_hbm,
                    [(embc_hbm, 0), (embc_hbm, 1),
                     (embx2_hbm, 0), (embx2_hbm, 1)], s2_o)


BN = 512
GRID = NPAD // BN


def _full(shape):
    return pl.BlockSpec(shape, lambda i: tuple(0 for _ in shape))


def _blk(shape, dim=0):
    def idx(i):
        return tuple(i if d == dim else 0 for d in range(len(shape)))
    return pl.BlockSpec(shape, idx)


def _tc1_body(f1, f2, wl1, wr1, wl2, wr2, xl1, xr1, xl2, xr2):
    a = f1[...]
    b = f2[...]
    xl1[...] = jnp.dot(a, wl1[...], preferred_element_type=F32)
    xr1[...] = jnp.dot(a, wr1[...], preferred_element_type=F32)
    xl2[...] = jnp.dot(b, wl2[...], preferred_element_type=F32)
    xr2[...] = jnp.dot(b, wr2[...], preferred_element_type=F32)


def _tc_xlxr(f1p, f2p, Wl1, Wr1, Wl2, Wr2):
    out = jax.ShapeDtypeStruct((NPAD, D_OUT), F32)
    return pl.pallas_call(
        _tc1_body,
        grid=(GRID,),
        in_specs=[_blk((BN, D_IN)), _blk((BN, D_IN)),
                  _full((D_IN, D_OUT)), _full((D_IN, D_OUT)),
                  _full((D_IN, D_OUT)), _full((D_IN, D_OUT))],
        out_specs=[_blk((BN, D_OUT))] * 4,
        out_shape=[out] * 4,
    )(f1p, f2p, Wl1, Wr1, Wl2, Wr2)


def _fin(accA, accB, ws, bias):
    a = accA[...]
    b = accB[...]
    den = jnp.sum(ws[...], axis=0) + 1e-16
    num = jnp.concatenate([a[0] + a[1], b[0] + b[1]], axis=-1)
    return num / den[:, None] + bias[...]


def _atten_blk(ea, eb, w_ref, u_ref):
    va = jnp.tanh(jnp.dot(ea, w_ref[...], preferred_element_type=F32))
    vb = jnp.tanh(jnp.dot(eb, w_ref[...], preferred_element_type=F32))
    u = u_ref[...]
    ua = jnp.dot(va, u, preferred_element_type=F32)[:, 0]
    ub = jnp.dot(vb, u, preferred_element_type=F32)[:, 0]
    m = jnp.maximum(ua, ub)
    x0 = jnp.exp(ua - m)
    x1 = jnp.exp(ub - m)
    den = x0 + x1
    a0 = x0 / den
    a1 = x1 / den
    comb = ea * a0[:, None] + eb * a1[:, None]
    return comb, jnp.stack([a0, a1], axis=1)


def _tc2_body(aA_sp1, aB_sp1, ws_sp1, aA_ft1, aB_ft1, ws_ft1,
              aA_sp2, aB_sp2, ws_sp2, aA_ft2, aB_ft2, ws_ft2,
              b1, b2, aw1, au1, aw2, au2, awc, auc,
              o_sp1, o_ft1, o_sp2, o_ft2, o_e1, o_e2, o_ec,
              o_al1, o_al2, o_al12):
    e_sp1 = _fin(aA_sp1, aB_sp1, ws_sp1, b1)
    e_ft1 = _fin(aA_ft1, aB_ft1, ws_ft1, b1)
    e_sp2 = _fin(aA_sp2, aB_sp2, ws_sp2, b2)
    e_ft2 = _fin(aA_ft2, aB_ft2, ws_ft2, b2)
    emb1, al1 = _atten_blk(e_sp1, e_ft1, aw1, au1)
    emb2, al2 = _atten_blk(e_sp2, e_ft2, aw2, au2)
    embc, al12 = _atten_blk(emb1, emb2, awc, auc)
    o_sp1[...] = e_sp1
    o_ft1[...] = e_ft1
    o_sp2[...] = e_sp2
    o_ft2[...] = e_ft2
    o_e1[...] = emb1
    o_e2[...] = emb2
    o_ec[...] = embc
    o_al1[...] = al1
    o_al2[...] = al2
    o_al12[...] = al12


def _tc_finalize1(aA_sp1, aB_sp1, ws_sp1, aA_ft1, aB_ft1, ws_ft1,
                  aA_sp2, aB_sp2, ws_sp2, aA_ft2, aB_ft2, ws_ft2,
                  b1, b2, aw1, au1, aw2, au2, awc, auc):
    acc_spec = _blk((NC, BN, H), dim=1)
    ws_spec = _blk((NW, BN), dim=1)
    emb = jax.ShapeDtypeStruct((NPAD, D_OUT), F32)
    al = jax.ShapeDtypeStruct((NPAD, 2), F32)
    return pl.pallas_call(
        _tc2_body,
        grid=(GRID,),
        in_specs=[acc_spec, acc_spec, ws_spec] * 4
        + [_full((1, D_OUT))] * 2
        + [_full((D_OUT, D_OUT)), _full((D_OUT, 1))] * 3,
        out_specs=[_blk((BN, D_OUT))] * 7 + [_blk((BN, 2))] * 3,
        out_shape=[emb] * 7 + [al] * 3,
    )(aA_sp1, aB_sp1, ws_sp1, aA_ft1, aB_ft1, ws_ft1,
      aA_sp2, aB_sp2, ws_sp2, aA_ft2, aB_ft2, ws_ft2,
      b1, b2, aw1, au1, aw2, au2, awc, auc)


def _tc3_body(s1, s2, dw1, dw2, wl1, wr1, wl2, wr2,
              o_rec1, o_rec2, o_xla1, o_xra1, o_xla2, o_xra2):
    t1 = s1[...]
    t2 = s2[...]
    sA = jnp.concatenate([t1[0], t1[1]], axis=-1)
    sB = jnp.concatenate([t1[2], t1[3]], axis=-1)
    sC = jnp.concatenate([t2[0], t2[1]], axis=-1)
    sD = jnp.concatenate([t2[2], t2[3]], axis=-1)
    w1 = dw1[...]
    w2 = dw2[...]
    o_rec1[...] = jnp.dot(sA, w1, preferred_element_type=F32)
    o_rec2[...] = jnp.dot(sC, w2, preferred_element_type=F32)
    x_a1 = jnp.dot(sD, w2, preferred_element_type=F32)
    x_a2 = jnp.dot(sB, w1, preferred_element_type=F32)
    o_xla1[...] = jnp.dot(x_a1, wl2[...], preferred_element_type=F32)
    o_xra1[...] = jnp.dot(x_a1, wr2[...], preferred_element_type=F32)
    o_xla2[...] = jnp.dot(x_a2, wl1[...], preferred_element_type=F32)
    o_xra2[...] = jnp.dot(x_a2, wr1[...], preferred_element_type=F32)


def _tc_stage3(s1, s2, dec_w1, dec_w2, Wl1, Wr1, Wl2, Wr2):
    rec = jax.ShapeDtypeStruct((NPAD, D_IN), F32)
    tab = jax.ShapeDtypeStruct((NPAD, D_OUT), F32)
    return pl.pallas_call(
        _tc3_body,
        grid=(GRID,),
        in_specs=[_blk((4, BN, H), dim=1), _blk((4, BN, H), dim=1),
                  _full((D_OUT, D_IN)), _full((D_OUT, D_IN)),
                  _full((D_IN, D_OUT)), _full((D_IN, D_OUT)),
                  _full((D_IN, D_OUT)), _full((D_IN, D_OUT))],
        out_specs=[_blk((BN, D_IN))] * 2 + [_blk((BN, D_OUT))] * 4,
        out_shape=[rec] * 2 + [tab] * 4,
    )(s1, s2, dec_w1, dec_w2, Wl1, Wr1, Wl2, Wr2)


def _tc4_body(aA1, aB1, ws1, aA2, aB2, ws2, b2, b1, o_a1, o_a2):
    o_a1[...] = _fin(aA1, aB1, ws1, b2)
    o_a2[...] = _fin(aA2, aB2, ws2, b1)


def _tc_finalize2(aA1, aB1, ws1, aA2, aB2, ws2, b2, b1):
    acc_spec = _blk((NC, BN, H), dim=1)
    ws_spec = _blk((NW, BN), dim=1)
    emb = jax.ShapeDtypeStruct((NPAD, D_OUT), F32)
    return pl.pallas_call(
        _tc4_body,
        grid=(GRID,),
        in_specs=[acc_spec, acc_spec, ws_spec] * 2 + [_full((1, D_OUT))] * 2,
        out_specs=[_blk((BN, D_OUT))] * 2,
        out_shape=[emb] * 2,
    )(aA1, aB1, ws1, aA2, aB2, ws2, b2, b1)


def _prep_conv_edges(e):
    loop = jnp.arange(N, dtype=I32)
    npad_e = ECP - ECONV
    src = jnp.concatenate([e[0], loop, jnp.zeros((npad_e,), I32)])
    dst = jnp.concatenate([e[1], loop, jnp.full((npad_e,), N, I32)])
    return src.reshape(ECP // 128, 128), dst.reshape(ECP // 128, 128)


def _prep_spmm_edges(e):
    npad_e = ESP - E
    row = jnp.concatenate([e[0], jnp.full((npad_e,), N, I32)])
    col = jnp.concatenate([e[1], jnp.zeros((npad_e,), I32)])
    return row.reshape(ESP // 128, 128), col.reshape(ESP // 128, 128)


def kernel(features_omics1, features_omics2, edge_index_spatial_omics1,
           edge_index_feature_omics1, edge_index_spatial_omics2,
           edge_index_feature_omics2, Wl1, Wr1, att1, b1, Wl2, Wr2, att2, b2,
           dec_w1, dec_w2, aw1, au1, aw2, au2, awc, auc):
    f1p = jnp.pad(features_omics1, ((0, NPAD - N), (0, 0)))
    f2p = jnp.pad(features_omics2, ((0, NPAD - N), (0, 0)))

    sp1_src, sp1_dst = _prep_conv_edges(edge_index_spatial_omics1)
    ft1_src, ft1_dst = _prep_conv_edges(edge_index_feature_omics1)
    sp2_src, sp2_dst = _prep_conv_edges(edge_index_spatial_omics2)
    ft2_src, ft2_dst = _prep_conv_edges(edge_index_feature_omics2)
    sp1_row, sp1_col = _prep_spmm_edges(edge_index_spatial_omics1)
    sp2_row, sp2_col = _prep_spmm_edges(edge_index_spatial_omics2)

    xl1, xr1, xl2, xr2 = _tc_xlxr(f1p, f2p, Wl1, Wr1, Wl2, Wr2)

    def conv(xl, xr, src, dst, att):
        accA, w = _sc_conv_pass_a(xl, xr, src, dst, att)
        accB = _sc_conv_pass_b(xl.reshape(2 * NPAD, H), src, dst, w)
        ws = _sc_conv_pass_c(dst, w)
        return accA, accB, ws

    aA_sp1, aB_sp1, ws_sp1 = conv(xl1, xr1, sp1_src, sp1_dst, att1)
    aA_ft1, aB_ft1, ws_ft1 = conv(xl1, xr1, ft1_src, ft1_dst, att1)
    aA_sp2, aB_sp2, ws_sp2 = conv(xl2, xr2, sp2_src, sp2_dst, att2)
    aA_ft2, aB_ft2, ws_ft2 = conv(xl2, xr2, ft2_src, ft2_dst, att2)

    (e_sp1, e_ft1, e_sp2, e_ft2, emb1, emb2, embc,
     al1, al2, al12) = _tc_finalize1(
        aA_sp1, aB_sp1, ws_sp1, aA_ft1, aB_ft1, ws_ft1,
        aA_sp2, aB_sp2, ws_sp2, aA_ft2, aB_ft2, ws_ft2,
        b1.reshape(1, D_OUT), b2.reshape(1, D_OUT),
        aw1, au1, aw2, au2, awc, auc)

    s1, s2 = _sc_spmm(embc.reshape(2 * NPAD, H), emb2.reshape(2 * NPAD, H),
                      emb1.reshape(2 * NPAD, H),
                      sp1_row, sp1_col, sp2_row, sp2_col)

    rec1, rec2, xla1, xra1, xla2, xra2 = _tc_stage3(
        s1, s2, dec_w1, dec_w2, Wl1, Wr1, Wl2, Wr2)

    aA_a1, aB_a1, ws_a1 = conv(xla1, xra1, sp2_src, sp2_dst, att2)
    aA_a2, aB_a2, ws_a2 = conv(xla2, xra2, sp1_src, sp1_dst, att1)

    across1, across2 = _tc_finalize2(aA_a1, aB_a1, ws_a1,
                                     aA_a2, aB_a2, ws_a2,
                                     b2.reshape(1, D_OUT),
                                     b1.reshape(1, D_OUT))

    return {
        'emb_latent_omics1': emb1[:N],
        'emb_latent_omics2': emb2[:N],
        'emb_latent_combined': embc[:N],
        'emb_recon_omics1': rec1[:N],
        'emb_recon_omics2': rec2[:N],
        'emb_latent_omics1_across_recon': across1[:N],
        'emb_latent_omics2_across_recon': across2[:N],
        'emb_latent_spatial_omics1': e_sp1[:N],
        'emb_latent_feature_omics1': e_ft1[:N],
        'emb_latent_spatial_omics2': e_sp2[:N],
        'emb_latent_feature_omics2': e_ft2[:N],
        'alpha_omics1': al1[:N],
        'alpha_omics2': al2[:N],
        'alpha': al12[:N],
    }
